# baseline (speedup 1.0000x reference)
import jax
import jax.numpy as jnp
from jax import lax
from jax.experimental import pallas as pl
from jax.experimental.pallas import tpu as pltpu

_NUM_EMB = 1024
_EMB_DIM = 64
_COMMIT = 0.25


def _vq_body(x_ref, w_ref, w2_ref, wsq_ref, iota_r_ref, iota_c_ref, enc_ref,
             qst_ref, loss_ref, ppl_ref, sse_ref, cnt_ref):
    i = pl.program_id(0)
    nsteps = pl.num_programs(0)
    w = w_ref[...]
    sse_part = jnp.zeros((), jnp.float32)
    cnt_part = jnp.zeros((1, _NUM_EMB), jnp.float32)
    hw = x_ref.shape[2]
    for k in range(x_ref.shape[0]):
        xb = x_ref[k]
        xsq = jnp.sum(xb * xb, axis=0, keepdims=True)
        m2 = lax.dot_general(w2_ref[...], xb, (((1,), (0,)), ((), ())),
                             preferred_element_type=jnp.float32)
        dist = (xsq + wsq_ref[...]) - m2
        dmin = jnp.min(dist, axis=0, keepdims=True)
        iota_c = iota_c_ref[...]
        idx_t = jnp.min(jnp.where(dist == dmin, iota_c, float(_NUM_EMB)),
                        axis=0, keepdims=True)
        idx = lax.transpose(idx_t, (1, 0))
        enc = (iota_r_ref[...] == idx).astype(jnp.float32)
        enc_ref[pl.ds(k * hw, hw), :] = enc
        q = lax.dot_general(w, enc, (((0,), (1,)), ((), ())),
                            preferred_element_type=jnp.float32)
        d = q - xb
        qst_ref[k] = xb + d
        sse_part += jnp.sum(d * d)
        ones_row = jnp.full((1, hw), 1.0, jnp.float32)
        cnt_part += lax.dot_general(ones_row, enc, (((1,), (0,)), ((), ())),
                                    preferred_element_type=jnp.float32)

    @pl.when(i == 0)
    def _init():
        sse_ref[0] = sse_part
        cnt_ref[...] = cnt_part

    @pl.when(i != 0)
    def _acc():
        sse_ref[0] += sse_part
        cnt_ref[...] += cnt_part

    @pl.when(i == nsteps - 1)
    def _fin():
        n_tok = nsteps * x_ref.shape[0] * hw
        mean = sse_ref[0] / (n_tok * _EMB_DIM)
        loss_ref[...] = jnp.reshape(mean + _COMMIT * mean, (1, 1))
        avg = cnt_ref[...] / n_tok
        ent = jnp.sum(avg * jnp.log(avg + 1e-10), axis=1, keepdims=True)
        ppl_ref[...] = jnp.exp(-ent)


def kernel(inputs, W):
    B, C, H, Wd = inputs.shape
    HW = H * Wd
    N = B * HW
    xv = inputs.reshape(B, C, HW)
    wsq = jnp.sum(W ** 2, axis=1).reshape(_NUM_EMB, 1)
    w2 = W + W
    iota_r = lax.broadcasted_iota(jnp.float32, (1, _NUM_EMB), 1)
    iota_c = lax.broadcasted_iota(jnp.float32, (_NUM_EMB, 1), 0)

    BB = 2
    enc, qst, loss, ppl = pl.pallas_call(
        _vq_body,
        grid=(B // BB,),
        in_specs=[
            pl.BlockSpec((BB, C, HW), lambda i: (i, 0, 0)),
            pl.BlockSpec((_NUM_EMB, C), lambda i: (0, 0)),
            pl.BlockSpec((_NUM_EMB, C), lambda i: (0, 0)),
            pl.BlockSpec((_NUM_EMB, 1), lambda i: (0, 0)),
            pl.BlockSpec((1, _NUM_EMB), lambda i: (0, 0)),
            pl.BlockSpec((_NUM_EMB, 1), lambda i: (0, 0)),
        ],
        out_specs=[
            pl.BlockSpec((BB * HW, _NUM_EMB), lambda i: (i, 0)),
            pl.BlockSpec((BB, C, HW), lambda i: (i, 0, 0)),
            pl.BlockSpec((1, 1), lambda i: (0, 0)),
            pl.BlockSpec((1, 1), lambda i: (0, 0)),
        ],
        out_shape=[
            jax.ShapeDtypeStruct((N, _NUM_EMB), jnp.float32),
            jax.ShapeDtypeStruct((B, C, HW), jnp.float32),
            jax.ShapeDtypeStruct((1, 1), jnp.float32),
            jax.ShapeDtypeStruct((1, 1), jnp.float32),
        ],
        scratch_shapes=[
            pltpu.SMEM((1,), jnp.float32),
            pltpu.VMEM((1, _NUM_EMB), jnp.float32),
        ],
    )(xv, W, w2, wsq, iota_r, iota_c)

    quantized_st = qst.reshape(B, C, H, Wd)
    return (loss[0, 0], quantized_st, ppl[0, 0], enc)
